# baseline (speedup 1.0000x reference)
import functools

import jax
import jax.numpy as jnp
from jax import lax
from jax.experimental import pallas as pl
from jax.experimental.pallas import tpu as pltpu
from jax.experimental.pallas import tpu_sc as plsc

EMBD = 64
NUM_CORES = 2
NUM_SUBCORES = 16
NW = NUM_CORES * NUM_SUBCORES
CHUNK = 512


@functools.partial(jax.jit, static_argnums=(2,))
def _gather_rows(idx, table, n_rows):
    per_w = n_rows // NW
    n_chunks = per_w // CHUNK
    mesh = plsc.VectorSubcoreMesh(core_axis_name="c", subcore_axis_name="s")

    @functools.partial(
        pl.kernel,
        mesh=mesh,
        compiler_params=pltpu.CompilerParams(use_tc_tiling_on_sc=False),
        out_type=jax.ShapeDtypeStruct((n_rows, EMBD), jnp.float32),
        scratch_types=[
            pltpu.VMEM((CHUNK,), jnp.int32),
            pltpu.VMEM((CHUNK, EMBD), jnp.float32),
            pltpu.SemaphoreType.DMA,
        ],
    )
    def body(idx_hbm, table_hbm, out_hbm, idx_v, rows_v, sem):
        wid = lax.axis_index("s") * NUM_CORES + lax.axis_index("c")
        base = wid * per_w

        def step(g, carry):
            off = base + g * CHUNK
            pltpu.sync_copy(idx_hbm.at[pl.ds(off, CHUNK)], idx_v)
            pltpu.async_copy(table_hbm.at[idx_v], rows_v, sem).wait()
            pltpu.sync_copy(rows_v, out_hbm.at[pl.ds(off, CHUNK)])
            return carry

        lax.fori_loop(0, n_chunks, step, 0)

    return body(idx, table)


def kernel(x, table):
    b, l = x.shape
    flat = x.reshape(-1).astype(jnp.int32)
    out = _gather_rows(flat, table, b * l)
    return out.reshape(b, l, EMBD)


# speedup vs baseline: 1.0421x; 1.0421x over previous
import functools

import jax
import jax.numpy as jnp
from jax import lax
from jax.experimental import pallas as pl
from jax.experimental.pallas import tpu as pltpu
from jax.experimental.pallas import tpu_sc as plsc

EMBD = 64
NUM_CORES = 2
NUM_SUBCORES = 16
NW = NUM_CORES * NUM_SUBCORES
CHUNK = 800


@functools.partial(jax.jit, static_argnums=(2,))
def _gather_rows(idx, table, n_rows):
    per_w = n_rows // NW
    n_chunks = per_w // CHUNK
    mesh = plsc.VectorSubcoreMesh(core_axis_name="c", subcore_axis_name="s")

    @functools.partial(
        pl.kernel,
        mesh=mesh,
        compiler_params=pltpu.CompilerParams(use_tc_tiling_on_sc=False),
        out_type=jax.ShapeDtypeStruct((n_rows, EMBD), jnp.float32),
        scratch_types=[
            pltpu.VMEM((per_w,), jnp.int32),
            pltpu.VMEM((2, CHUNK, EMBD), jnp.float32),
            pltpu.SemaphoreType.DMA((2,)),
            pltpu.SemaphoreType.DMA((2,)),
        ],
    )
    def body(idx_hbm, table_hbm, out_hbm, idx_v, rows_v, gsem, ssem):
        wid = lax.axis_index("s") * NUM_CORES + lax.axis_index("c")
        base = wid * per_w
        pltpu.sync_copy(idx_hbm.at[pl.ds(base, per_w)], idx_v)

        def gather_desc(g, p):
            return pltpu.make_async_copy(
                table_hbm.at[idx_v.at[pl.ds(g * CHUNK, CHUNK)]],
                rows_v.at[p],
                gsem.at[p],
            )

        def store_desc(g, p):
            return pltpu.make_async_copy(
                rows_v.at[p],
                out_hbm.at[pl.ds(base + g * CHUNK, CHUNK)],
                ssem.at[p],
            )

        gather_desc(0, 0).start()
        gather_desc(0, 0).wait()
        gather_desc(1, 1).start()
        store_desc(0, 0).start()

        def step(g, carry):
            p = g % 2
            q = 1 - p
            gather_desc(g, p).wait()
            store_desc(g - 1, q).wait()
            gather_desc(g + 1, q).start()
            store_desc(g, p).start()
            return carry

        lax.fori_loop(1, n_chunks - 1, step, 0)

        g = n_chunks - 1
        p = g % 2
        gather_desc(g, p).wait()
        store_desc(g - 1, 1 - p).wait()
        store_desc(g, p).start()
        store_desc(g, p).wait()

    return body(idx, table)


def kernel(x, table):
    b, l = x.shape
    flat = x.reshape(-1).astype(jnp.int32)
    out = _gather_rows(flat, table, b * l)
    return out.reshape(b, l, EMBD)
